# baseline (speedup 1.0000x reference)
import functools

import jax
import jax.numpy as jnp
from jax import lax
from jax.experimental import pallas as pl
from jax.experimental.pallas import tpu as pltpu
from jax.experimental.pallas import tpu_sc as plsc

_NUM_FIELDS = 26
_VOCAB = 100000
_EMBED_DIM = 32
_BATCH = 16384


_OUT_ROWS = _VOCAB * _EMBED_DIM // 128
_NFULL = _VOCAB // 128
_TAIL_V = _VOCAB - _NFULL * 128


def _pack4(blk, rows):
    a = blk.T.reshape(rows, 4, _EMBED_DIM)
    return jnp.concatenate([a[:, c, :] for c in range(4)], axis=1)


def _transpose_body(tab_ref, out_ref):
    def chunk(c, carry):
        v0 = pl.multiple_of(c * 128, 128)
        p0 = pl.multiple_of(c * 32, 32)
        out_ref[pl.ds(p0, 32), :] = _pack4(tab_ref[:, pl.ds(v0, 128)], 32)
        return carry

    lax.fori_loop(0, _NFULL, chunk, 0)
    out_ref[pl.ds(_NFULL * 32, _TAIL_V // 4), :] = _pack4(
        tab_ref[:, pl.ds(_NFULL * 128, _TAIL_V)], _TAIL_V // 4
    )


_to_rowmajor = pl.pallas_call(
    _transpose_body,
    grid=(_NUM_FIELDS,),
    in_specs=[
        pl.BlockSpec((_EMBED_DIM, _VOCAB), lambda f: (f, 0)),
    ],
    out_specs=pl.BlockSpec((_OUT_ROWS, 128), lambda f: (f, 0)),
    out_shape=jax.ShapeDtypeStruct(
        (_NUM_FIELDS * _VOCAB * _EMBED_DIM // 128, 128), jnp.float32
    ),
)


_NC = 2
_NS = 16
_NW = _NC * _NS
_N = _BATCH * _NUM_FIELDS
_ROWS_W = _N // _NW
_CHUNK = 1024
_NCHUNK = _ROWS_W // _CHUNK
_NBUF = 2


def _gather_body(idx_hbm, tab_hbm, out_hbm, idx_v, rows_v, sem_g, sem_w):
    wid = lax.axis_index("s") * _NC + lax.axis_index("c")
    pltpu.sync_copy(idx_hbm.at[pl.ds(wid * _NCHUNK, _NCHUNK)], idx_v)

    def fire(step, b):
        pltpu.async_copy(
            tab_hbm.at[idx_v.at[step]], rows_v.at[b], sem_g.at[b]
        )

    fire(0, 0)

    @pl.loop(0, _NCHUNK)
    def _step(g):
        b = lax.rem(g, _NBUF)
        nb = lax.rem(g + 1, _NBUF)

        @pl.when(g + 1 < _NCHUNK)
        def _prefetch():
            @pl.when(g + 1 >= _NBUF)
            def _reclaim():
                pltpu.make_async_copy(
                    rows_v.at[nb],
                    out_hbm.at[pl.ds(0, _CHUNK)],
                    sem_w.at[nb],
                ).wait()

            fire(g + 1, nb)

        pltpu.make_async_copy(
            tab_hbm.at[pl.ds(0, _CHUNK)], rows_v.at[b], sem_g.at[b]
        ).wait()
        pltpu.async_copy(
            rows_v.at[b],
            out_hbm.at[pl.ds(wid * _ROWS_W + g * _CHUNK, _CHUNK)],
            sem_w.at[b],
        )

    for b in range(_NBUF):
        pltpu.make_async_copy(
            rows_v.at[b], out_hbm.at[pl.ds(0, _CHUNK)], sem_w.at[b]
        ).wait()


_gather = functools.partial(
    pl.kernel,
    out_type=jax.ShapeDtypeStruct((_N, _EMBED_DIM), jnp.float32),
    mesh=plsc.VectorSubcoreMesh(
        core_axis_name="c", subcore_axis_name="s",
        num_cores=_NC, num_subcores=_NS,
    ),
    scratch_types=[
        pltpu.VMEM((_NCHUNK, _CHUNK), jnp.int32),
        pltpu.VMEM((_NBUF, _CHUNK, _EMBED_DIM), jnp.float32),
        pltpu.SemaphoreType.DMA((_NBUF,)),
        pltpu.SemaphoreType.DMA((_NBUF,)),
    ],
    compiler_params=pltpu.CompilerParams(use_tc_tiling_on_sc=False),
)(_gather_body)


def kernel(x, tables):
    tab_t = jnp.transpose(tables, (0, 2, 1))
    tab_2d = tab_t.reshape(_NUM_FIELDS * _EMBED_DIM, _VOCAB)
    tab_rm = _to_rowmajor(tab_2d)
    tab_flat = tab_rm.reshape(_NUM_FIELDS * _VOCAB, _EMBED_DIM)

    offs = jnp.arange(_NUM_FIELDS, dtype=jnp.int32) * _VOCAB
    flat_idx = (x.astype(jnp.int32) + offs[None, :]).reshape(
        _N // _CHUNK, _CHUNK
    )
    out = _gather(flat_idx, tab_flat)
    return out.reshape(_BATCH, _NUM_FIELDS, _EMBED_DIM)


# speedup vs baseline: 3.5879x; 3.5879x over previous
import functools

import jax
import jax.numpy as jnp
from jax import lax
from jax.experimental import pallas as pl
from jax.experimental.pallas import tpu as pltpu
from jax.experimental.pallas import tpu_sc as plsc

_NUM_FIELDS = 26
_VOCAB = 100000
_EMBED_DIM = 32
_BATCH = 16384

_NC = 2
_NS = 16
_NBUF = 2


def _gather_body(x_hbm, tab_hbm, out_hbm, idx_v, row_v, sem_i, sem_g, sem_w):
    e = lax.axis_index("s") * _NC + lax.axis_index("c")

    pltpu.async_copy(x_hbm.at[0], idx_v.at[0], sem_i.at[0])

    @pl.loop(0, _NUM_FIELDS)
    def _field(f):
        b = lax.rem(f, _NBUF)
        nb = lax.rem(f + 1, _NBUF)

        @pl.when(f + 1 < _NUM_FIELDS)
        def _prefetch_idx():
            pltpu.async_copy(x_hbm.at[f + 1], idx_v.at[nb], sem_i.at[nb])

        pltpu.make_async_copy(x_hbm.at[0], idx_v.at[b], sem_i.at[b]).wait()

        @pl.when(f >= _NBUF)
        def _reclaim():
            pltpu.make_async_copy(
                row_v.at[b], out_hbm.at[0, 0], sem_w.at[b]
            ).wait()

        pltpu.async_copy(
            tab_hbm.at[f, e].at[idx_v.at[b]], row_v.at[b], sem_g.at[b]
        )
        pltpu.make_async_copy(
            tab_hbm.at[0, 0], row_v.at[b], sem_g.at[b]
        ).wait()
        pltpu.async_copy(row_v.at[b], out_hbm.at[f, e], sem_w.at[b])

    for b in range(_NBUF):
        pltpu.make_async_copy(
            row_v.at[b], out_hbm.at[0, 0], sem_w.at[b]
        ).wait()


def _make_gather(nfields):
    return functools.partial(
        pl.kernel,
        out_type=jax.ShapeDtypeStruct(
            (nfields, _EMBED_DIM, _BATCH), jnp.float32
        ),
        mesh=plsc.VectorSubcoreMesh(
            core_axis_name="c", subcore_axis_name="s",
            num_cores=_NC, num_subcores=_NS,
        ),
        scratch_types=[
            pltpu.VMEM((_NBUF, _BATCH), jnp.int32),
            pltpu.VMEM((_NBUF, _BATCH), jnp.float32),
            pltpu.SemaphoreType.DMA((_NBUF,)),
            pltpu.SemaphoreType.DMA((_NBUF,)),
            pltpu.SemaphoreType.DMA((_NBUF,)),
        ],
        compiler_params=pltpu.CompilerParams(use_tc_tiling_on_sc=False),
    )(_gather_body)


_gather = _make_gather(_NUM_FIELDS)


def kernel(x, tables):
    x_t = x.T.astype(jnp.int32)
    tab_t = jnp.transpose(tables, (0, 2, 1))
    out_t = _gather(x_t, tab_t)
    return jnp.transpose(out_t, (2, 0, 1))
